# baseline (speedup 1.0000x reference)
import functools

import jax
import jax.numpy as jnp
from jax import lax
from jax.experimental import pallas as pl
from jax.experimental.pallas import tpu as pltpu
from jax.experimental.pallas import tpu_sc as plsc

GAMMA = 2.0


_ROWS_PER_BLOCK = 512
_N_STREAMS = 8


def _rowstats_body(*refs):
    xs = refs[:_N_STREAMS]
    labs = refs[_N_STREAMS:2 * _N_STREAMS]
    lz_ref, xl_ref = refs[-2], refs[-1]
    r = xs[0].shape[0]
    for k in range(_N_STREAMS):
        x = xs[k][...]
        lab = labs[k][...]
        m = jnp.max(x, axis=1, keepdims=True)
        s = jnp.sum(jnp.exp(x - m), axis=1, keepdims=True)
        cls = lax.broadcasted_iota(jnp.int32, x.shape, 1)
        xl = jnp.sum(jnp.where(cls == lab, x, 0.0), axis=1, keepdims=True)
        lz_ref[pl.ds(k * r, r), :] = m + jnp.log(s)
        xl_ref[pl.ds(k * r, r), :] = xl


def _rowstats(logits, labels_col):
    b, c = logits.shape
    r = _ROWS_PER_BLOCK
    ns = _N_STREAMS
    nblk = b // (r * ns)
    x_specs = [
        pl.BlockSpec((r, c), functools.partial(lambda k, i: (i * ns + k, 0), k))
        for k in range(ns)
    ]
    lab_specs = [
        pl.BlockSpec((r, 1), functools.partial(lambda k, i: (i * ns + k, 0), k))
        for k in range(ns)
    ]
    out_sds = jax.ShapeDtypeStruct((b, 1), jnp.float32)
    return pl.pallas_call(
        _rowstats_body,
        grid=(nblk,),
        in_specs=x_specs + lab_specs,
        out_specs=[pl.BlockSpec((r * ns, 1), lambda i: (i, 0))] * 2,
        out_shape=[out_sds, out_sds],
    )(*([logits] * ns + [labels_col] * ns))


_LANES = 16
_CHUNK = 128


def _sc_alpha_body(nc, chunks_per_w, labels_hbm, alpha_hbm, al_hbm,
                   lab_v, al_v, sem):
    wid = lax.axis_index("s") * nc + lax.axis_index("c")
    row0 = wid * chunks_per_w
    pltpu.sync_copy(labels_hbm.at[pl.ds(row0, chunks_per_w)], lab_v)
    copies = []
    for j in range(chunks_per_w):
        copies.append(
            pltpu.async_copy(alpha_hbm.at[lab_v.at[j]], al_v.at[j], sem))
    for cp in copies:
        cp.wait()
    pltpu.sync_copy(al_v, al_hbm.at[pl.ds(row0, chunks_per_w)])


def _sc_alpha_gather(labels2d, alpha_flat):
    nrows, _ = labels2d.shape
    info = plsc.get_sparse_core_info()
    nc, ns = info.num_cores, info.num_subcores
    nw = nc * ns
    chunks_per_w = nrows // nw
    mesh = plsc.VectorSubcoreMesh(core_axis_name="c", subcore_axis_name="s")
    out_sds = jax.ShapeDtypeStruct((nrows, _CHUNK), jnp.float32)
    k = pl.kernel(
        functools.partial(_sc_alpha_body, nc, chunks_per_w),
        mesh=mesh,
        out_type=[out_sds],
        scratch_types=[
            pltpu.VMEM((chunks_per_w, _CHUNK), jnp.int32),
            pltpu.VMEM((chunks_per_w, _CHUNK), jnp.float32),
            pltpu.SemaphoreType.DMA,
        ],
    )
    return k(labels2d, alpha_flat)


def _combine_body(xl_ref, al_ref, lz_ref, out_ref):
    logp = xl_ref[...] - lz_ref[...]
    p = jnp.exp(logp)
    q = 1.0 - p
    loss = -al_ref[...] * q * q * logp
    out_ref[...] = (jnp.sum(loss) / loss.size).reshape(1, 1)


def _combine(xl2d, al2d, lz2d):
    return pl.pallas_call(
        _combine_body,
        out_shape=jax.ShapeDtypeStruct((1, 1), jnp.float32),
    )(xl2d, al2d, lz2d)


def kernel(logits, labels, alpha):
    b, c = logits.shape
    labels2d = labels.reshape(b // _CHUNK, _CHUNK).astype(jnp.int32)
    (al2d,) = _sc_alpha_gather(labels2d, alpha.reshape(-1))
    lz, xl = _rowstats(logits, labels2d.reshape(b, 1))
    sh = (b // _CHUNK, _CHUNK)
    loss = _combine(xl.reshape(sh), al2d, lz.reshape(sh))
    return loss[0, 0]


# speedup vs baseline: 1.0059x; 1.0059x over previous
import functools

import jax
import jax.numpy as jnp
from jax import lax
from jax.experimental import pallas as pl
from jax.experimental.pallas import tpu as pltpu
from jax.experimental.pallas import tpu_sc as plsc

GAMMA = 2.0


_ROWS_PER_BLOCK = 512
_N_STREAMS = 4


def _rowstats_body(*refs):
    xs = refs[:_N_STREAMS]
    labs = refs[_N_STREAMS:2 * _N_STREAMS]
    lz_ref, xl_ref = refs[-2], refs[-1]
    r = xs[0].shape[0]
    for k in range(_N_STREAMS):
        x = xs[k][...]
        lab = labs[k][...]
        m = jnp.max(x, axis=1, keepdims=True)
        s = jnp.sum(jnp.exp(x - m), axis=1, keepdims=True)
        cls = lax.broadcasted_iota(jnp.int32, x.shape, 1)
        xl = jnp.sum(jnp.where(cls == lab, x, 0.0), axis=1, keepdims=True)
        lz_ref[pl.ds(k * r, r), :] = m + jnp.log(s)
        xl_ref[pl.ds(k * r, r), :] = xl


def _rowstats(logits, labels_col):
    b, c = logits.shape
    r = _ROWS_PER_BLOCK
    ns = _N_STREAMS
    nblk = b // (r * ns)
    x_specs = [
        pl.BlockSpec((r, c), functools.partial(lambda k, i: (i * ns + k, 0), k))
        for k in range(ns)
    ]
    lab_specs = [
        pl.BlockSpec((r, 1), functools.partial(lambda k, i: (i * ns + k, 0), k))
        for k in range(ns)
    ]
    out_sds = jax.ShapeDtypeStruct((b, 1), jnp.float32)
    return pl.pallas_call(
        _rowstats_body,
        grid=(nblk,),
        in_specs=x_specs + lab_specs,
        out_specs=[pl.BlockSpec((r * ns, 1), lambda i: (i, 0))] * 2,
        out_shape=[out_sds, out_sds],
    )(*([logits] * ns + [labels_col] * ns))


_LANES = 16
_CHUNK = 128


def _sc_alpha_body(nc, chunks_per_w, labels_hbm, alpha_hbm, al_hbm,
                   lab_v, al_v, sem):
    wid = lax.axis_index("s") * nc + lax.axis_index("c")
    row0 = wid * chunks_per_w
    pltpu.sync_copy(labels_hbm.at[pl.ds(row0, chunks_per_w)], lab_v)
    copies = []
    for j in range(chunks_per_w):
        copies.append(
            pltpu.async_copy(alpha_hbm.at[lab_v.at[j]], al_v.at[j], sem))
    for cp in copies:
        cp.wait()
    pltpu.sync_copy(al_v, al_hbm.at[pl.ds(row0, chunks_per_w)])


def _sc_alpha_gather(labels2d, alpha_flat):
    nrows, _ = labels2d.shape
    info = plsc.get_sparse_core_info()
    nc, ns = info.num_cores, info.num_subcores
    nw = nc * ns
    chunks_per_w = nrows // nw
    mesh = plsc.VectorSubcoreMesh(core_axis_name="c", subcore_axis_name="s")
    out_sds = jax.ShapeDtypeStruct((nrows, _CHUNK), jnp.float32)
    k = pl.kernel(
        functools.partial(_sc_alpha_body, nc, chunks_per_w),
        mesh=mesh,
        out_type=[out_sds],
        scratch_types=[
            pltpu.VMEM((chunks_per_w, _CHUNK), jnp.int32),
            pltpu.VMEM((chunks_per_w, _CHUNK), jnp.float32),
            pltpu.SemaphoreType.DMA,
        ],
    )
    return k(labels2d, alpha_flat)


def _combine_body(xl_ref, al_ref, lz_ref, out_ref):
    logp = xl_ref[...] - lz_ref[...]
    p = jnp.exp(logp)
    q = 1.0 - p
    loss = -al_ref[...] * q * q * logp
    out_ref[...] = (jnp.sum(loss) / loss.size).reshape(1, 1)


def _combine(xl2d, al2d, lz2d):
    return pl.pallas_call(
        _combine_body,
        out_shape=jax.ShapeDtypeStruct((1, 1), jnp.float32),
    )(xl2d, al2d, lz2d)


def kernel(logits, labels, alpha):
    b, c = logits.shape
    labels2d = labels.reshape(b // _CHUNK, _CHUNK).astype(jnp.int32)
    (al2d,) = _sc_alpha_gather(labels2d, alpha.reshape(-1))
    lz, xl = _rowstats(logits, labels2d.reshape(b, 1))
    sh = (b // _CHUNK, _CHUNK)
    loss = _combine(xl.reshape(sh), al2d, lz.reshape(sh))
    return loss[0, 0]


# speedup vs baseline: 2.3691x; 2.3552x over previous
import functools

import jax
import jax.numpy as jnp
from jax import lax
from jax.experimental import pallas as pl
from jax.experimental.pallas import tpu as pltpu
from jax.experimental.pallas import tpu_sc as plsc

GAMMA = 2.0


_COLS_PER_BLOCK = 512
_N_STREAMS = 4


def _colstats_body(*refs):
    xs = refs[:_N_STREAMS]
    labs = refs[_N_STREAMS:2 * _N_STREAMS]
    lz_ref, xl_ref = refs[-2], refs[-1]
    w = xs[0].shape[1]
    for k in range(_N_STREAMS):
        x = xs[k][...]
        lab = labs[k][...]
        m = jnp.max(x, axis=0, keepdims=True)
        s = jnp.sum(jnp.exp(x - m), axis=0, keepdims=True)
        cls = lax.broadcasted_iota(jnp.int32, x.shape, 0)
        xl = jnp.sum(jnp.where(cls == lab, x, 0.0), axis=0, keepdims=True)
        lz_ref[:, pl.ds(k * w, w)] = m + jnp.log(s)
        xl_ref[:, pl.ds(k * w, w)] = xl


def _colstats(logits_t, labels_row):
    c, b = logits_t.shape
    w = _COLS_PER_BLOCK
    ns = _N_STREAMS
    nblk = b // (w * ns)
    x_specs = [
        pl.BlockSpec((c, w), functools.partial(lambda k, i: (0, i * ns + k), k))
        for k in range(ns)
    ]
    lab_specs = [
        pl.BlockSpec((1, w), functools.partial(lambda k, i: (0, i * ns + k), k))
        for k in range(ns)
    ]
    out_sds = jax.ShapeDtypeStruct((1, b), jnp.float32)
    return pl.pallas_call(
        _colstats_body,
        grid=(nblk,),
        in_specs=x_specs + lab_specs,
        out_specs=[pl.BlockSpec((1, w * ns), lambda i: (0, i))] * 2,
        out_shape=[out_sds, out_sds],
    )(*([logits_t] * ns + [labels_row] * ns))


_LANES = 16
_CHUNK = 128


def _sc_alpha_body(nc, chunks_per_w, labels_hbm, alpha_hbm, al_hbm,
                   lab_v, al_v, sem):
    wid = lax.axis_index("s") * nc + lax.axis_index("c")
    row0 = wid * chunks_per_w
    pltpu.sync_copy(labels_hbm.at[pl.ds(row0, chunks_per_w)], lab_v)
    copies = []
    for j in range(chunks_per_w):
        copies.append(
            pltpu.async_copy(alpha_hbm.at[lab_v.at[j]], al_v.at[j], sem))
    for cp in copies:
        cp.wait()
    pltpu.sync_copy(al_v, al_hbm.at[pl.ds(row0, chunks_per_w)])


def _sc_alpha_gather(labels2d, alpha_flat):
    nrows, _ = labels2d.shape
    info = plsc.get_sparse_core_info()
    nc, ns = info.num_cores, info.num_subcores
    nw = nc * ns
    chunks_per_w = nrows // nw
    mesh = plsc.VectorSubcoreMesh(core_axis_name="c", subcore_axis_name="s")
    out_sds = jax.ShapeDtypeStruct((nrows, _CHUNK), jnp.float32)
    k = pl.kernel(
        functools.partial(_sc_alpha_body, nc, chunks_per_w),
        mesh=mesh,
        out_type=[out_sds],
        scratch_types=[
            pltpu.VMEM((chunks_per_w, _CHUNK), jnp.int32),
            pltpu.VMEM((chunks_per_w, _CHUNK), jnp.float32),
            pltpu.SemaphoreType.DMA,
        ],
    )
    return k(labels2d, alpha_flat)


def _combine_body(xl_ref, al_ref, lz_ref, out_ref):
    logp = xl_ref[...] - lz_ref[...]
    p = jnp.exp(logp)
    q = 1.0 - p
    loss = -al_ref[...] * q * q * logp
    out_ref[...] = (jnp.sum(loss) / loss.size).reshape(1, 1)


def _combine(xl_row, al_row, lz_row):
    return pl.pallas_call(
        _combine_body,
        out_shape=jax.ShapeDtypeStruct((1, 1), jnp.float32),
    )(xl_row, al_row, lz_row)


def kernel(logits, labels, alpha):
    b, c = logits.shape
    labels = labels.astype(jnp.int32)
    labels2d = labels.reshape(b // _CHUNK, _CHUNK)
    (al2d,) = _sc_alpha_gather(labels2d, alpha.reshape(-1))
    lz, xl = _colstats(logits.T, labels.reshape(1, b))
    loss = _combine(xl, al2d.reshape(1, b), lz)
    return loss[0, 0]
